# Initial kernel scaffold; baseline (speedup 1.0000x reference)
import jax
import jax.numpy as jnp
from jax.experimental import pallas as pl


def kernel(X, B, L, L_u, L_d, W1, W2, W3, Wfc):
    raise NotImplementedError("write your pallas kernel here")


# speedup vs baseline: 1.3559x; 1.3559x over previous
import functools

import jax
import jax.numpy as jnp
from jax.experimental import pallas as pl
from jax.experimental.pallas import tpu as pltpu


def _split_bf16(a):
    hi = a.astype(jnp.bfloat16)
    lo = (a - hi.astype(jnp.float32)).astype(jnp.bfloat16)
    return hi, lo


def _dot3(a, b, dims):
    ahi, alo = _split_bf16(a)
    bhi, blo = _split_bf16(b)

    def d(x, y):
        return jax.lax.dot_general(x, y, dims,
                                   preferred_element_type=jnp.float32)

    return d(ahi, bhi) + (d(ahi, blo) + d(alo, bhi))


def _dual_body(b_ref, xq_ref, xp_ref, q_ref, p_ref):
    i = pl.program_id(1)
    b = b_ref[0]
    q_ref[0] = jnp.dot(b, xq_ref[0], preferred_element_type=jnp.float32)
    p_c = jax.lax.dot_general(
        b, xp_ref[0], (((0,), (0,)), ((), ())),
        preferred_element_type=jnp.float32)

    @pl.when(i == 0)
    def _():
        p_ref[0] = p_c

    @pl.when(i != 0)
    def _():
        p_ref[0] += p_c


def _dual_products(Bm, Xin, block_rows):
    _, N, _ = Bm.shape
    C = Xin.shape[2]
    T = block_rows
    grid = (2, N // T)
    out_shape = [jax.ShapeDtypeStruct((2, N, C), jnp.float32)] * 2
    return pl.pallas_call(
        _dual_body,
        grid=grid,
        in_specs=[
            pl.BlockSpec((1, T, N), lambda m, i: (m, i, 0)),
            pl.BlockSpec((1, N, C), lambda m, i: (m + 1, 0, 0)),
            pl.BlockSpec((1, T, C), lambda m, i: (m, i, 0)),
        ],
        out_specs=[
            pl.BlockSpec((1, T, C), lambda m, i: (m, i, 0)),
            pl.BlockSpec((1, N, C), lambda m, i: (m, 0, 0)),
        ],
        out_shape=out_shape,
        compiler_params=pltpu.CompilerParams(
            vmem_limit_bytes=128 * 1024 * 1024),
    )(Bm, Xin, Xin)


def _combine_body(apply_fc, *refs):
    if apply_fc:
        x_ref, q_ref, p_ref, w_ref, fc_ref, y_ref = refs
    else:
        x_ref, q_ref, p_ref, w_ref, y_ref = refs
    d = pl.program_id(0)
    acc = x_ref[0]
    acc = acc + jnp.where(d > 0, p_ref[0], 0.0)
    acc = acc + jnp.where(d < 2, q_ref[0], 0.0)
    y = jnp.tanh(jnp.dot(acc, w_ref[0], preferred_element_type=jnp.float32))
    if apply_fc:
        y = jnp.dot(y, fc_ref[...], preferred_element_type=jnp.float32)
    y_ref[0] = y


def _combine(Xin, Q, P, W, WfcT, block_rows):
    _, N, C = Xin.shape
    F = W.shape[2]
    Tc = block_rows
    grid = (3, N // Tc)
    in_specs = [
        pl.BlockSpec((1, Tc, C), lambda d, i: (d, i, 0)),
        pl.BlockSpec((1, Tc, C), lambda d, i: (jnp.minimum(d, 1), i, 0)),
        pl.BlockSpec((1, Tc, C), lambda d, i: (jnp.maximum(d - 1, 0), i, 0)),
        pl.BlockSpec((1, C, F), lambda d, i: (d, 0, 0)),
    ]
    args = [Xin, Q, P, W]
    if WfcT is not None:
        in_specs.append(pl.BlockSpec((F, F), lambda d, i: (0, 0)))
        args.append(WfcT)
    return pl.pallas_call(
        functools.partial(_combine_body, WfcT is not None),
        grid=grid,
        in_specs=in_specs,
        out_specs=pl.BlockSpec((1, Tc, F), lambda d, i: (d, i, 0)),
        out_shape=jax.ShapeDtypeStruct((3, N, F), jnp.float32),
    )(*args)


def kernel(X, B, L, L_u, L_d, W1, W2, W3, Wfc):
    del L, L_u, L_d
    WfcT = Wfc.T
    Xcur = X
    for W, fc in ((W1, None), (W2, None), (W3, WfcT)):
        Q, P = _dual_products(B, Xcur, block_rows=256)
        Xcur = _combine(Xcur, Q, P, W, fc, block_rows=2048)
    return Xcur
